# baseline (speedup 1.0000x reference)
import functools

import jax
import jax.numpy as jnp
from jax import lax
from jax.experimental import pallas as pl
from jax.experimental.pallas import tpu as pltpu
from jax.experimental.pallas import tpu_sc as plsc

N = 10000
E = 160000
ET = E + N
EP = 172032
IN_DIM = 256
HIDDEN = 128
S_DIM = 64
P_DIM = 64
ETA = 1e-6
NEG_SLOPE = 0.2
SLICE_W = 128
NACC = 10112
ROWS_PER_TILE = NACC // 16

def _mesh():
    return plsc.VectorSubcoreMesh(core_axis_name="c", subcore_axis_name="s")


def _pack_bf16_pairs(v):
    h = v.shape[1] // 2
    lo = v[:, :h].astype(jnp.bfloat16).astype(jnp.float32)
    hi = v[:, h:].astype(jnp.bfloat16).astype(jnp.float32)
    lo_u = lax.bitcast_convert_type(lo, jnp.uint32)
    hi_u = lax.bitcast_convert_type(hi, jnp.uint32)
    packed = lax.shift_right_logical(lo_u, jnp.uint32(16)) | (hi_u & jnp.uint32(0xFFFF0000))
    return lax.bitcast_convert_type(packed, jnp.int32)


def _unpack_bf16_pairs(gi):
    u = lax.bitcast_convert_type(gi, jnp.uint32)
    lo = lax.bitcast_convert_type(lax.shift_left(u, jnp.uint32(16)), jnp.float32)
    hi = lax.bitcast_convert_type(u & jnp.uint32(0xFFFF0000), jnp.float32)
    return jnp.concatenate([lo, hi], axis=1)


def _mm2_body(x_ref, wl_ref, bl_ref, wr_ref, br_ref, xl_ref, xr_ref, *, packed):
    x = x_ref[...]
    xl = lax.dot_general(x, wl_ref[...], (((1,), (0,)), ((), ())),
                         preferred_element_type=jnp.float32) + bl_ref[...]
    xr = lax.dot_general(x, wr_ref[...], (((1,), (0,)), ((), ())),
                         preferred_element_type=jnp.float32) + br_ref[...]
    if packed:
        xl_ref[...] = _pack_bf16_pairs(xl)
        xr_ref[...] = _pack_bf16_pairs(xr)
    else:
        xl_ref[...] = xl
        xr_ref[...] = xr


def _project(x, wl, bl, wr, br, packed):
    n, in_dim = x.shape
    hd = wl.shape[1]
    wout = hd // 2 if packed else hd
    odt = jnp.int32 if packed else jnp.float32
    bm = 2000
    return pl.pallas_call(
        functools.partial(_mm2_body, packed=packed),
        grid=(n // bm,),
        in_specs=[
            pl.BlockSpec((bm, in_dim), lambda i: (i, 0)),
            pl.BlockSpec((in_dim, hd), lambda i: (0, 0)),
            pl.BlockSpec((1, hd), lambda i: (0, 0)),
            pl.BlockSpec((in_dim, hd), lambda i: (0, 0)),
            pl.BlockSpec((1, hd), lambda i: (0, 0)),
        ],
        out_specs=(
            pl.BlockSpec((bm, wout), lambda i: (i, 0)),
            pl.BlockSpec((bm, wout), lambda i: (i, 0)),
        ),
        out_shape=(
            jax.ShapeDtypeStruct((n, wout), odt),
            jax.ShapeDtypeStruct((n, wout), odt),
        ),
    )(x, wl, bl.reshape(1, hd), wr, br.reshape(1, hd))


def _gather_rows(xl, xr, src1, dst1, chunk):
    hd = xl.shape[1]
    per_tile = EP // 32
    nch = per_tile // chunk

    @functools.partial(
        pl.kernel,
        mesh=_mesh(),
        out_type=(
            jax.ShapeDtypeStruct((EP, hd), xl.dtype),
            jax.ShapeDtypeStruct((EP, hd), xl.dtype),
        ),
        scratch_types=[
            pltpu.VMEM((2, chunk), jnp.int32),
            pltpu.VMEM((2, chunk), jnp.int32),
            pltpu.VMEM((2, chunk, hd), xl.dtype),
            pltpu.VMEM((2, chunk, hd), xl.dtype),
            pltpu.SemaphoreType.DMA((2,)),
            pltpu.SemaphoreType.DMA((2,)),
            pltpu.SemaphoreType.DMA((2,)),
            pltpu.SemaphoreType.DMA((2,)),
        ],
    )
    def k(xl_hbm, xr_hbm, src_hbm, dst_hbm, gl_hbm, gr_hbm,
          is_v, id_v, gl_v, gr_v, gsem1, gsem2, osem1, osem2):
        wid = lax.axis_index("s") * 2 + lax.axis_index("c")
        base = wid * per_tile

        def start(ci, b):
            off = base + ci * chunk
            pltpu.sync_copy(src_hbm.at[pl.ds(off, chunk)], is_v.at[b])
            pltpu.sync_copy(dst_hbm.at[pl.ds(off, chunk)], id_v.at[b])
            pltpu.async_copy(xl_hbm.at[is_v.at[b]], gl_v.at[b], gsem1.at[b])
            pltpu.async_copy(xr_hbm.at[id_v.at[b]], gr_v.at[b], gsem2.at[b])

        def finish(ci, b):
            off = base + ci * chunk
            pltpu.make_async_copy(xl_hbm.at[is_v.at[b]], gl_v.at[b], gsem1.at[b]).wait()
            pltpu.make_async_copy(xr_hbm.at[id_v.at[b]], gr_v.at[b], gsem2.at[b]).wait()
            pltpu.async_copy(gl_v.at[b], gl_hbm.at[pl.ds(off, chunk)], osem1.at[b])
            pltpu.async_copy(gr_v.at[b], gr_hbm.at[pl.ds(off, chunk)], osem2.at[b])

        def drain(ci, b):
            off = base + ci * chunk
            pltpu.make_async_copy(gl_v.at[b], gl_hbm.at[pl.ds(off, chunk)], osem1.at[b]).wait()
            pltpu.make_async_copy(gr_v.at[b], gr_hbm.at[pl.ds(off, chunk)], osem2.at[b]).wait()

        start(0, 0)

        @pl.loop(0, nch - 1)
        def _(ci):
            b = lax.rem(ci, 2)
            nb = 1 - b
            @pl.when(ci >= 1)
            def _():
                drain(ci - 1, nb)
            start(ci + 1, nb)
            finish(ci, b)

        last = nch - 1
        finish_b = lax.rem(last, 2)
        drain(last - 1, 1 - finish_b)
        finish(last, finish_b)
        drain(last, finish_b)

    return k(xl, xr, src1, dst1)


def _alpha_body(gl_ref, gr_ref, attb_ref, alpha_ref, gmax_ref, *, be, packed):
    i = pl.program_id(0)
    if packed:
        t = _unpack_bf16_pairs(gl_ref[...]) + _unpack_bf16_pairs(gr_ref[...])
    else:
        t = gl_ref[...] + gr_ref[...]
    t = jnp.where(t > 0, t, NEG_SLOPE * t)
    a = lax.dot_general(t, attb_ref[...], (((1,), (0,)), ((), ())),
                        preferred_element_type=jnp.float32)
    rows = i * be + lax.broadcasted_iota(jnp.int32, a.shape, 0)
    a = jnp.where(rows < ET, a, -1e30)
    alpha_ref[...] = a
    bm8 = jnp.broadcast_to(jnp.max(a, axis=0, keepdims=True), (8, 16))

    @pl.when(i == 0)
    def _():
        gmax_ref[...] = bm8

    @pl.when(i > 0)
    def _():
        gmax_ref[...] = jnp.maximum(gmax_ref[...], bm8)


def _edge_alpha(gl, gr, attb, packed):
    hw = gl.shape[1]
    be = 1024
    return pl.pallas_call(
        functools.partial(_alpha_body, be=be, packed=packed),
        grid=(EP // be,),
        in_specs=[
            pl.BlockSpec((be, hw), lambda i: (i, 0)),
            pl.BlockSpec((be, hw), lambda i: (i, 0)),
            pl.BlockSpec((attb.shape[0], 16), lambda i: (0, 0)),
        ],
        out_specs=(
            pl.BlockSpec((be, 16), lambda i: (i, 0)),
            pl.BlockSpec((8, 16), lambda i: (0, 0)),
        ),
        out_shape=(
            jax.ShapeDtypeStruct((EP, 16), jnp.float32),
            jax.ShapeDtypeStruct((8, 16), jnp.float32),
        ),
    )(gl, gr, attb)


def _msg_body(gl_ref, alpha_ref, gmax_ref, *m_refs, heads, packed):
    g = jnp.max(gmax_ref[...], axis=0, keepdims=True)
    w = jnp.exp(alpha_ref[...] - g)
    if packed:
        gl = _unpack_bf16_pairs(gl_ref[...])
    else:
        gl = gl_ref[...]
    for q in range(heads):
        m_refs[q][...] = gl[:, q * 128:(q + 1) * 128] * w[:, q:q + 1]
    pad = jnp.zeros((w.shape[0], SLICE_W - 16), jnp.float32)
    m_refs[heads][...] = jnp.concatenate([w, pad], axis=1)


def _edge_messages(gl, alpha, gmax, heads, packed):
    hw = gl.shape[1]
    be = 1024
    return pl.pallas_call(
        functools.partial(_msg_body, heads=heads, packed=packed),
        grid=(EP // be,),
        in_specs=[
            pl.BlockSpec((be, hw), lambda i: (i, 0)),
            pl.BlockSpec((be, 16), lambda i: (i, 0)),
            pl.BlockSpec((8, 16), lambda i: (0, 0)),
        ],
        out_specs=tuple(pl.BlockSpec((be, SLICE_W), lambda i: (i, 0))
                        for _ in range(heads + 1)),
        out_shape=tuple(jax.ShapeDtypeStruct((EP, SLICE_W), jnp.float32)
                        for _ in range(heads + 1)),
    )(gl, alpha, gmax)


def _scatter_accumulate(dst1, m_slices, zeros_nw):
    nsl = len(m_slices)
    win = 128

    per_tile = EP // 32
    nw = per_tile // win

    @functools.partial(
        pl.kernel,
        mesh=_mesh(),
        out_type=jax.ShapeDtypeStruct((nsl, 2, NACC, SLICE_W), jnp.float32),
        scratch_types=[
            pltpu.VMEM_SHARED((NACC, SLICE_W), jnp.float32),
            pltpu.VMEM((2, win), jnp.int32),
            pltpu.VMEM((2, win, SLICE_W), jnp.float32),
            pltpu.SemaphoreType.DMA((2,)),
            pltpu.SemaphoreType.DMA((2,)),
        ],
    )
    def k(dst_hbm, *rest):
        m_hbms = rest[:nsl]
        z_hbm = rest[nsl]
        p_hbm = rest[nsl + 1]
        acc, id_v, rows_v, rsem, asem = rest[nsl + 2:]
        cid = lax.axis_index("c")
        sid = lax.axis_index("s")
        row0 = sid * ROWS_PER_TILE
        base = (sid * 2 + cid) * per_tile

        for q in range(nsl):
            m_hbm = m_hbms[q]

            def start(ci, b, m_hbm=m_hbm):
                off = base + ci * win
                pltpu.sync_copy(dst_hbm.at[pl.ds(off, win)], id_v.at[b])
                pltpu.async_copy(m_hbm.at[pl.ds(off, win)], rows_v.at[b],
                                 rsem.at[b])

            def add(ci, b, m_hbm=m_hbm):
                off = base + ci * win
                pltpu.make_async_copy(m_hbm.at[pl.ds(off, win)], rows_v.at[b],
                                      rsem.at[b]).wait()
                pltpu.async_copy(rows_v.at[b], acc.at[id_v.at[b]], asem.at[b])

            def drain(b):
                pltpu.make_async_copy(rows_v.at[b], acc.at[id_v.at[b]],
                                      asem.at[b]).wait()

            pltpu.sync_copy(z_hbm.at[pl.ds(row0, ROWS_PER_TILE)],
                            acc.at[pl.ds(row0, ROWS_PER_TILE)])
            plsc.subcore_barrier()
            start(0, 0)

            @pl.loop(0, nw - 1)
            def _(ci):
                b = lax.rem(ci, 2)
                nb = 1 - b

                @pl.when(ci >= 1)
                def _():
                    drain(nb)

                start(ci + 1, nb)
                add(ci, b)

            last_b = lax.rem(nw - 1, 2)
            drain(1 - last_b)
            add(nw - 1, last_b)
            drain(last_b)
            plsc.subcore_barrier()
            pltpu.sync_copy(acc.at[pl.ds(row0, ROWS_PER_TILE)],
                            p_hbm.at[q, cid, pl.ds(row0, ROWS_PER_TILE)])
            plsc.subcore_barrier()

    return k(dst1, *m_slices, zeros_nw)


def _norm_body(p_ref, b_ref, o_ref, *, heads, relu):
    p = p_ref[...]
    ps = p[:, 0] + p[:, 1]
    den = ps[heads, :, :heads]
    parts = [ps[h] / (den[:, h:h + 1] + 1e-30) for h in range(heads)]
    out = jnp.concatenate(parts, axis=1) + b_ref[...]
    o_ref[...] = jnp.maximum(out, 0.0) if relu else out


def _normalize(p, bias, heads, relu):
    bn = 1000
    hd = heads * 128
    return pl.pallas_call(
        functools.partial(_norm_body, heads=heads, relu=relu),
        grid=(N // bn,),
        in_specs=[
            pl.BlockSpec((heads + 1, 2, bn, SLICE_W), lambda i: (0, 0, i, 0)),
            pl.BlockSpec((1, hd), lambda i: (0, 0)),
        ],
        out_specs=pl.BlockSpec((bn, hd), lambda i: (i, 0)),
        out_shape=jax.ShapeDtypeStruct((N, hd), jnp.float32),
    )(p, bias.reshape(1, hd))


def _gat_layer(x, wl, bl, wr, br, att, bias, heads, src1, dst1, zeros_nw, relu):
    hd = heads * 128
    attb = jnp.zeros((hd, 16), jnp.float32)
    for h in range(heads):
        attb = attb.at[h * 128:(h + 1) * 128, h].set(att[h])

    packed = hd == 512
    xl, xr = _project(x, wl, bl, wr, br, packed)
    chunk = 112 if packed else 128
    gl, gr = _gather_rows(xl, xr, src1, dst1, chunk)
    alpha, gmax = _edge_alpha(gl, gr, attb, packed)
    m_slices = _edge_messages(gl, alpha, gmax, heads, packed)
    p = _scatter_accumulate(dst1, m_slices, zeros_nw)
    return _normalize(p, bias, heads, relu)


def _head_body(s_ref, p_ref, o1_ref, o2_ref):
    s = s_ref[...]
    p = p_ref[...]
    o1_ref[...] = jnp.concatenate([s[:, :S_DIM], p[:, :P_DIM]], axis=-1)
    sp_s = jnp.logaddexp(s[:, S_DIM:], 0.0)
    sp_p = jnp.logaddexp(p[:, P_DIM:], 0.0)
    o2_ref[...] = jnp.concatenate([sp_s + ETA, sp_p + ETA], axis=-1)


def kernel(x, edge_index, Wl_s1, bl_s1, Wr_s1, br_s1, att_s1, b_s1,
           Wl_s2, bl_s2, Wr_s2, br_s2, att_s2, b_s2,
           Wl_p1, bl_p1, Wr_p1, br_p1, att_p1, b_p1,
           Wl_p2, bl_p2, Wr_p2, br_p2, att_p2, b_p2):
    loop = jnp.arange(N, dtype=edge_index.dtype)
    pad = jnp.zeros((EP - ET,), edge_index.dtype)
    src1 = jnp.concatenate([edge_index[0], loop, pad])
    dst1 = jnp.concatenate([edge_index[1], loop, pad])
    zeros_nw = jnp.zeros((NACC, SLICE_W), jnp.float32)

    s = _gat_layer(x, Wl_s1, bl_s1, Wr_s1, br_s1, att_s1, b_s1, 4,
                   src1, dst1, zeros_nw, relu=True)
    p = _gat_layer(x, Wl_p1, bl_p1, Wr_p1, br_p1, att_p1, b_p1, 4,
                   src1, dst1, zeros_nw, relu=False)
    s = _gat_layer(s, Wl_s2, bl_s2, Wr_s2, br_s2, att_s2, b_s2, 1,
                   src1, dst1, zeros_nw, relu=False)
    p = _gat_layer(p, Wl_p2, bl_p2, Wr_p2, br_p2, att_p2, b_p2, 1,
                   src1, dst1, zeros_nw, relu=False)

    out1, out2 = pl.pallas_call(
        _head_body,
        out_shape=(
            jax.ShapeDtypeStruct((N, S_DIM + P_DIM), jnp.float32),
            jax.ShapeDtypeStruct((N, S_DIM + P_DIM), jnp.float32),
        ),
    )(s, p)
    return (out1, out2)


# speedup vs baseline: 1.0447x; 1.0447x over previous
import functools

import jax
import jax.numpy as jnp
from jax import lax
from jax.experimental import pallas as pl
from jax.experimental.pallas import tpu as pltpu
from jax.experimental.pallas import tpu_sc as plsc

N = 10000
E = 160000
ET = E + N
EP = 172032
IN_DIM = 256
HIDDEN = 128
S_DIM = 64
P_DIM = 64
ETA = 1e-6
NEG_SLOPE = 0.2
SLICE_W = 128
NACC = 10112
ROWS_PER_TILE = NACC // 16

def _mesh():
    return plsc.VectorSubcoreMesh(core_axis_name="c", subcore_axis_name="s")


def _pack_bf16_pairs(v):
    h = v.shape[1] // 2
    lo = v[:, :h].astype(jnp.bfloat16).astype(jnp.float32)
    hi = v[:, h:].astype(jnp.bfloat16).astype(jnp.float32)
    lo_u = lax.bitcast_convert_type(lo, jnp.uint32)
    hi_u = lax.bitcast_convert_type(hi, jnp.uint32)
    packed = lax.shift_right_logical(lo_u, jnp.uint32(16)) | (hi_u & jnp.uint32(0xFFFF0000))
    return lax.bitcast_convert_type(packed, jnp.int32)


def _unpack_bf16_pairs(gi):
    u = lax.bitcast_convert_type(gi, jnp.uint32)
    lo = lax.bitcast_convert_type(lax.shift_left(u, jnp.uint32(16)), jnp.float32)
    hi = lax.bitcast_convert_type(u & jnp.uint32(0xFFFF0000), jnp.float32)
    return jnp.concatenate([lo, hi], axis=1)


def _mm2_body(xa_ref, xb_ref, wla_ref, bla_ref, wra_ref, bra_ref,
              wlb_ref, blb_ref, wrb_ref, brb_ref, xl_ref, xr_ref, *, packed):
    xa = xa_ref[...]
    xb = xb_ref[...]

    def proj(x, w_ref, b_ref):
        return lax.dot_general(x, w_ref[...], (((1,), (0,)), ((), ())),
                               preferred_element_type=jnp.float32) + b_ref[...]

    xla = proj(xa, wla_ref, bla_ref)
    xra = proj(xa, wra_ref, bra_ref)
    xlb = proj(xb, wlb_ref, blb_ref)
    xrb = proj(xb, wrb_ref, brb_ref)
    if packed:
        xla, xra, xlb, xrb = (_pack_bf16_pairs(v) for v in (xla, xra, xlb, xrb))
    xl_ref[...] = jnp.concatenate([xla, xlb], axis=1)
    xr_ref[...] = jnp.concatenate([xra, xrb], axis=1)


def _project_pair(xa, xb, wa, xw_b, packed):
    n = xa.shape[0]
    ina, inb = xa.shape[1], xb.shape[1]
    hd = wa[0].shape[1]
    half = hd // 2 if packed else hd
    wout = 2 * half
    odt = jnp.int32 if packed else jnp.float32
    bm = 2000
    wla, bla, wra, bra = wa
    wlb, blb, wrb, brb = xw_b
    return pl.pallas_call(
        functools.partial(_mm2_body, packed=packed),
        grid=(n // bm,),
        in_specs=[
            pl.BlockSpec((bm, ina), lambda i: (i, 0)),
            pl.BlockSpec((bm, inb), lambda i: (i, 0)),
            pl.BlockSpec((ina, hd), lambda i: (0, 0)),
            pl.BlockSpec((1, hd), lambda i: (0, 0)),
            pl.BlockSpec((ina, hd), lambda i: (0, 0)),
            pl.BlockSpec((1, hd), lambda i: (0, 0)),
            pl.BlockSpec((inb, hd), lambda i: (0, 0)),
            pl.BlockSpec((1, hd), lambda i: (0, 0)),
            pl.BlockSpec((inb, hd), lambda i: (0, 0)),
            pl.BlockSpec((1, hd), lambda i: (0, 0)),
        ],
        out_specs=(
            pl.BlockSpec((bm, wout), lambda i: (i, 0)),
            pl.BlockSpec((bm, wout), lambda i: (i, 0)),
        ),
        out_shape=(
            jax.ShapeDtypeStruct((n, wout), odt),
            jax.ShapeDtypeStruct((n, wout), odt),
        ),
    )(xa, xb, wla, bla.reshape(1, hd), wra, bra.reshape(1, hd),
      wlb, blb.reshape(1, hd), wrb, brb.reshape(1, hd))


def _gather_rows(xl, xr, src1, dst1, chunk):
    hd = xl.shape[1]
    per_tile = EP // 32
    nch = per_tile // chunk

    @functools.partial(
        pl.kernel,
        mesh=_mesh(),
        out_type=(
            jax.ShapeDtypeStruct((EP, hd), xl.dtype),
            jax.ShapeDtypeStruct((EP, hd), xl.dtype),
        ),
        scratch_types=[
            pltpu.VMEM((2, chunk), jnp.int32),
            pltpu.VMEM((2, chunk), jnp.int32),
            pltpu.VMEM((2, chunk, hd), xl.dtype),
            pltpu.VMEM((2, chunk, hd), xl.dtype),
            pltpu.SemaphoreType.DMA((2,)),
            pltpu.SemaphoreType.DMA((2,)),
            pltpu.SemaphoreType.DMA((2,)),
            pltpu.SemaphoreType.DMA((2,)),
        ],
    )
    def k(xl_hbm, xr_hbm, src_hbm, dst_hbm, gl_hbm, gr_hbm,
          is_v, id_v, gl_v, gr_v, gsem1, gsem2, osem1, osem2):
        wid = lax.axis_index("s") * 2 + lax.axis_index("c")
        base = wid * per_tile

        def start(ci, b):
            off = base + ci * chunk
            pltpu.sync_copy(src_hbm.at[pl.ds(off, chunk)], is_v.at[b])
            pltpu.sync_copy(dst_hbm.at[pl.ds(off, chunk)], id_v.at[b])
            pltpu.async_copy(xl_hbm.at[is_v.at[b]], gl_v.at[b], gsem1.at[b])
            pltpu.async_copy(xr_hbm.at[id_v.at[b]], gr_v.at[b], gsem2.at[b])

        def finish(ci, b):
            off = base + ci * chunk
            pltpu.make_async_copy(xl_hbm.at[is_v.at[b]], gl_v.at[b], gsem1.at[b]).wait()
            pltpu.make_async_copy(xr_hbm.at[id_v.at[b]], gr_v.at[b], gsem2.at[b]).wait()
            pltpu.async_copy(gl_v.at[b], gl_hbm.at[pl.ds(off, chunk)], osem1.at[b])
            pltpu.async_copy(gr_v.at[b], gr_hbm.at[pl.ds(off, chunk)], osem2.at[b])

        def drain(ci, b):
            off = base + ci * chunk
            pltpu.make_async_copy(gl_v.at[b], gl_hbm.at[pl.ds(off, chunk)], osem1.at[b]).wait()
            pltpu.make_async_copy(gr_v.at[b], gr_hbm.at[pl.ds(off, chunk)], osem2.at[b]).wait()

        start(0, 0)

        @pl.loop(0, nch - 1)
        def _(ci):
            b = lax.rem(ci, 2)
            nb = 1 - b
            @pl.when(ci >= 1)
            def _():
                drain(ci - 1, nb)
            start(ci + 1, nb)
            finish(ci, b)

        last = nch - 1
        finish_b = lax.rem(last, 2)
        drain(last - 1, 1 - finish_b)
        finish(last, finish_b)
        drain(last, finish_b)

    return k(xl, xr, src1, dst1)


def _alpha_body(gl_ref, gr_ref, attb_ref, alpha_ref, gmax_ref, *, be, packed):
    i = pl.program_id(0)
    if packed:
        t = _unpack_bf16_pairs(gl_ref[...]) + _unpack_bf16_pairs(gr_ref[...])
    else:
        t = gl_ref[...] + gr_ref[...]
    t = jnp.where(t > 0, t, NEG_SLOPE * t)
    a = lax.dot_general(t, attb_ref[...], (((1,), (0,)), ((), ())),
                        preferred_element_type=jnp.float32)
    rows = i * be + lax.broadcasted_iota(jnp.int32, a.shape, 0)
    a = jnp.where(rows < ET, a, -1e30)
    alpha_ref[...] = a
    bm8 = jnp.broadcast_to(jnp.max(a, axis=0, keepdims=True), (8, 16))

    @pl.when(i == 0)
    def _():
        gmax_ref[...] = bm8

    @pl.when(i > 0)
    def _():
        gmax_ref[...] = jnp.maximum(gmax_ref[...], bm8)


def _edge_alpha(gl, gr, attb, packed, col):
    hw = gl.shape[1] // 2
    be = 1024
    return pl.pallas_call(
        functools.partial(_alpha_body, be=be, packed=packed),
        grid=(EP // be,),
        in_specs=[
            pl.BlockSpec((be, hw), lambda i, col=col: (i, col)),
            pl.BlockSpec((be, hw), lambda i, col=col: (i, col)),
            pl.BlockSpec((attb.shape[0], 16), lambda i: (0, 0)),
        ],
        out_specs=(
            pl.BlockSpec((be, 16), lambda i: (i, 0)),
            pl.BlockSpec((8, 16), lambda i: (0, 0)),
        ),
        out_shape=(
            jax.ShapeDtypeStruct((EP, 16), jnp.float32),
            jax.ShapeDtypeStruct((8, 16), jnp.float32),
        ),
    )(gl, gr, attb)


def _msg_body(gl_ref, alpha_ref, gmax_ref, *m_refs, heads, packed):
    g = jnp.max(gmax_ref[...], axis=0, keepdims=True)
    w = jnp.exp(alpha_ref[...] - g)
    if packed:
        gl = _unpack_bf16_pairs(gl_ref[...])
    else:
        gl = gl_ref[...]
    for q in range(heads):
        m_refs[q][...] = gl[:, q * 128:(q + 1) * 128] * w[:, q:q + 1]
    pad = jnp.zeros((w.shape[0], SLICE_W - 16), jnp.float32)
    m_refs[heads][...] = jnp.concatenate([w, pad], axis=1)


def _edge_messages(gl, alpha, gmax, heads, packed, col):
    hw = gl.shape[1] // 2
    be = 1024
    return pl.pallas_call(
        functools.partial(_msg_body, heads=heads, packed=packed),
        grid=(EP // be,),
        in_specs=[
            pl.BlockSpec((be, hw), lambda i, col=col: (i, col)),
            pl.BlockSpec((be, 16), lambda i: (i, 0)),
            pl.BlockSpec((8, 16), lambda i: (0, 0)),
        ],
        out_specs=tuple(pl.BlockSpec((be, SLICE_W), lambda i: (i, 0))
                        for _ in range(heads + 1)),
        out_shape=tuple(jax.ShapeDtypeStruct((EP, SLICE_W), jnp.float32)
                        for _ in range(heads + 1)),
    )(gl, alpha, gmax)


def _scatter_accumulate(dst2, m_slices, zeros_nw):
    nsl = len(m_slices)
    win = 128

    @functools.partial(
        pl.kernel,
        mesh=_mesh(),
        out_type=jax.ShapeDtypeStruct((nsl, 2, NACC, SLICE_W), jnp.float32),
        scratch_types=[pltpu.VMEM_SHARED((NACC, SLICE_W), jnp.float32)],
    )
    def k(dst_hbm, *rest):
        m_hbms = rest[:nsl]
        z_hbm = rest[nsl]
        p_hbm = rest[nsl + 1]
        acc = rest[nsl + 2]
        cid = lax.axis_index("c")
        sid = lax.axis_index("s")
        row0 = sid * ROWS_PER_TILE

        def body(id_v, rows_v):
            pltpu.sync_copy(rows_v, acc.at[id_v.at[0]], add=True)

        for q in range(nsl):
            pltpu.sync_copy(z_hbm.at[pl.ds(row0, ROWS_PER_TILE)],
                            acc.at[pl.ds(row0, ROWS_PER_TILE)])
            plsc.subcore_barrier()
            pltpu.emit_pipeline(
                body,
                grid=(EP // win,),
                in_specs=[
                    pl.BlockSpec((1, win), lambda i: (0, i)),
                    pl.BlockSpec((win, SLICE_W), lambda i: (i, 0)),
                ],
                out_specs=[],
                core_axis_name=("c", "s"),
                dimension_semantics=(pltpu.PARALLEL,),
            )(dst_hbm, m_hbms[q])
            plsc.subcore_barrier()
            pltpu.sync_copy(acc.at[pl.ds(row0, ROWS_PER_TILE)],
                            p_hbm.at[q, cid, pl.ds(row0, ROWS_PER_TILE)])
            plsc.subcore_barrier()

    return k(dst2, *m_slices, zeros_nw)


def _norm_body(p_ref, b_ref, o_ref, *, heads, relu):
    p = p_ref[...]
    ps = p[:, 0] + p[:, 1]
    den = ps[heads, :, :heads]
    parts = [ps[h] / (den[:, h:h + 1] + 1e-30) for h in range(heads)]
    out = jnp.concatenate(parts, axis=1) + b_ref[...]
    o_ref[...] = jnp.maximum(out, 0.0) if relu else out


def _normalize(p, bias, heads, relu, qoff):
    bn = 1000
    hd = heads * 128
    return pl.pallas_call(
        functools.partial(_norm_body, heads=heads, relu=relu),
        grid=(N // bn,),
        in_specs=[
            pl.BlockSpec((heads + 1, 2, bn, SLICE_W),
                         lambda i, qoff=qoff: (qoff, 0, i, 0)),
            pl.BlockSpec((1, hd), lambda i: (0, 0)),
        ],
        out_specs=pl.BlockSpec((bn, hd), lambda i: (i, 0)),
        out_shape=jax.ShapeDtypeStruct((N, hd), jnp.float32),
    )(p, bias.reshape(1, hd))


def _attb(att, heads):
    hd = heads * 128
    m = jnp.zeros((hd, 16), jnp.float32)
    for h in range(heads):
        m = m.at[h * 128:(h + 1) * 128, h].set(att[h])
    return m


def _gat_pair(xa, xb, wa, wb, atta, attb_, ba, bb, heads, src1, dst1, dst2,
              zeros_nw, relu_a):
    packed = heads == 4
    xl, xr = _project_pair(xa, xb, wa, wb, packed)
    chunk = 56 if packed else 112
    gl, gr = _gather_rows(xl, xr, src1, dst1, chunk)
    ma = _attb(atta, heads)
    mb = _attb(attb_, heads)
    alpha_a, gmax_a = _edge_alpha(gl, gr, ma, packed, 0)
    alpha_b, gmax_b = _edge_alpha(gl, gr, mb, packed, 1)
    msl_a = _edge_messages(gl, alpha_a, gmax_a, heads, packed, 0)
    msl_b = _edge_messages(gl, alpha_b, gmax_b, heads, packed, 1)
    p = _scatter_accumulate(dst2, list(msl_a) + list(msl_b), zeros_nw)
    out_a = _normalize(p, ba, heads, relu_a, 0)
    out_b = _normalize(p, bb, heads, False, 1)
    return out_a, out_b


def _head_body(s_ref, p_ref, o1_ref, o2_ref):
    s = s_ref[...]
    p = p_ref[...]
    o1_ref[...] = jnp.concatenate([s[:, :S_DIM], p[:, :P_DIM]], axis=-1)
    sp_s = jnp.logaddexp(s[:, S_DIM:], 0.0)
    sp_p = jnp.logaddexp(p[:, P_DIM:], 0.0)
    o2_ref[...] = jnp.concatenate([sp_s + ETA, sp_p + ETA], axis=-1)


def kernel(x, edge_index, Wl_s1, bl_s1, Wr_s1, br_s1, att_s1, b_s1,
           Wl_s2, bl_s2, Wr_s2, br_s2, att_s2, b_s2,
           Wl_p1, bl_p1, Wr_p1, br_p1, att_p1, b_p1,
           Wl_p2, bl_p2, Wr_p2, br_p2, att_p2, b_p2):
    loop = jnp.arange(N, dtype=edge_index.dtype)
    pad = jnp.zeros((EP - ET,), edge_index.dtype)
    src1 = jnp.concatenate([edge_index[0], loop, pad])
    dst1 = jnp.concatenate([edge_index[1], loop, pad])
    dst2 = dst1.reshape(1, EP)
    zeros_nw = jnp.zeros((NACC, SLICE_W), jnp.float32)

    s, p = _gat_pair(x, x, (Wl_s1, bl_s1, Wr_s1, br_s1),
                     (Wl_p1, bl_p1, Wr_p1, br_p1), att_s1, att_p1,
                     b_s1, b_p1, 4, src1, dst1, dst2, zeros_nw, relu_a=True)
    s, p = _gat_pair(s, p, (Wl_s2, bl_s2, Wr_s2, br_s2),
                     (Wl_p2, bl_p2, Wr_p2, br_p2), att_s2, att_p2,
                     b_s2, b_p2, 1, src1, dst1, dst2, zeros_nw, relu_a=False)

    out1, out2 = pl.pallas_call(
        _head_body,
        out_shape=(
            jax.ShapeDtypeStruct((N, S_DIM + P_DIM), jnp.float32),
            jax.ShapeDtypeStruct((N, S_DIM + P_DIM), jnp.float32),
        ),
    )(s, p)
    return (out1, out2)


# speedup vs baseline: 1.4269x; 1.3659x over previous
import functools

import jax
import jax.numpy as jnp
from jax import lax
from jax.experimental import pallas as pl
from jax.experimental.pallas import tpu as pltpu
from jax.experimental.pallas import tpu_sc as plsc

N = 10000
E = 160000
ET = E + N
EP = 172032
IN_DIM = 256
HIDDEN = 128
S_DIM = 64
P_DIM = 64
ETA = 1e-6
NEG_SLOPE = 0.2
SLICE_W = 128
NACC = 10112
ROWS_PER_TILE = NACC // 16

def _mesh():
    return plsc.VectorSubcoreMesh(core_axis_name="c", subcore_axis_name="s")


def _pack_bf16_pairs(v):
    h = v.shape[1] // 2
    lo = v[:, :h].astype(jnp.bfloat16).astype(jnp.float32)
    hi = v[:, h:].astype(jnp.bfloat16).astype(jnp.float32)
    lo_u = lax.bitcast_convert_type(lo, jnp.uint32)
    hi_u = lax.bitcast_convert_type(hi, jnp.uint32)
    packed = lax.shift_right_logical(lo_u, jnp.uint32(16)) | (hi_u & jnp.uint32(0xFFFF0000))
    return lax.bitcast_convert_type(packed, jnp.int32)


def _unpack_bf16_pairs(gi):
    u = lax.bitcast_convert_type(gi, jnp.uint32)
    lo = lax.bitcast_convert_type(lax.shift_left(u, jnp.uint32(16)), jnp.float32)
    hi = lax.bitcast_convert_type(u & jnp.uint32(0xFFFF0000), jnp.float32)
    return jnp.concatenate([lo, hi], axis=1)


def _mm2_body(xa_ref, xb_ref, wla_ref, bla_ref, wra_ref, bra_ref,
              wlb_ref, blb_ref, wrb_ref, brb_ref, xl_ref, xr_ref, *, packed):
    xa = xa_ref[...]
    xb = xb_ref[...]

    def proj(x, w_ref, b_ref):
        return lax.dot_general(x, w_ref[...], (((1,), (0,)), ((), ())),
                               preferred_element_type=jnp.float32) + b_ref[...]

    xla = proj(xa, wla_ref, bla_ref)
    xra = proj(xa, wra_ref, bra_ref)
    xlb = proj(xb, wlb_ref, blb_ref)
    xrb = proj(xb, wrb_ref, brb_ref)
    if packed == "each":
        xl_ref[...] = jnp.concatenate([_pack_bf16_pairs(xla), _pack_bf16_pairs(xlb)], axis=1)
        xr_ref[...] = jnp.concatenate([_pack_bf16_pairs(xra), _pack_bf16_pairs(xrb)], axis=1)
    else:
        xl_ref[...] = _pack_bf16_pairs(jnp.concatenate([xla, xlb], axis=1))
        xr_ref[...] = _pack_bf16_pairs(jnp.concatenate([xra, xrb], axis=1))


def _project_pair(xa, xb, wa, xw_b, packed):
    n = xa.shape[0]
    ina, inb = xa.shape[1], xb.shape[1]
    hd = wa[0].shape[1]
    wout = hd
    odt = jnp.int32
    bm = 2000
    wla, bla, wra, bra = wa
    wlb, blb, wrb, brb = xw_b
    return pl.pallas_call(
        functools.partial(_mm2_body, packed=packed),
        grid=(n // bm,),
        in_specs=[
            pl.BlockSpec((bm, ina), lambda i: (i, 0)),
            pl.BlockSpec((bm, inb), lambda i: (i, 0)),
            pl.BlockSpec((ina, hd), lambda i: (0, 0)),
            pl.BlockSpec((1, hd), lambda i: (0, 0)),
            pl.BlockSpec((ina, hd), lambda i: (0, 0)),
            pl.BlockSpec((1, hd), lambda i: (0, 0)),
            pl.BlockSpec((inb, hd), lambda i: (0, 0)),
            pl.BlockSpec((1, hd), lambda i: (0, 0)),
            pl.BlockSpec((inb, hd), lambda i: (0, 0)),
            pl.BlockSpec((1, hd), lambda i: (0, 0)),
        ],
        out_specs=(
            pl.BlockSpec((bm, wout), lambda i: (i, 0)),
            pl.BlockSpec((bm, wout), lambda i: (i, 0)),
        ),
        out_shape=(
            jax.ShapeDtypeStruct((n, wout), odt),
            jax.ShapeDtypeStruct((n, wout), odt),
        ),
    )(xa, xb, wla, bla.reshape(1, hd), wra, bra.reshape(1, hd),
      wlb, blb.reshape(1, hd), wrb, brb.reshape(1, hd))


def _gather_rows(xl, xr, src1, dst1, chunk):
    hd = xl.shape[1]
    per_tile = EP // 32
    nch = per_tile // chunk

    @functools.partial(
        pl.kernel,
        mesh=_mesh(),
        out_type=(
            jax.ShapeDtypeStruct((EP, hd), xl.dtype),
            jax.ShapeDtypeStruct((EP, hd), xl.dtype),
        ),
        scratch_types=[
            pltpu.VMEM((2, chunk), jnp.int32),
            pltpu.VMEM((2, chunk), jnp.int32),
            pltpu.VMEM((2, chunk, hd), xl.dtype),
            pltpu.VMEM((2, chunk, hd), xl.dtype),
            pltpu.SemaphoreType.DMA((2,)),
            pltpu.SemaphoreType.DMA((2,)),
            pltpu.SemaphoreType.DMA((2,)),
            pltpu.SemaphoreType.DMA((2,)),
        ],
    )
    def k(xl_hbm, xr_hbm, src_hbm, dst_hbm, gl_hbm, gr_hbm,
          is_v, id_v, gl_v, gr_v, gsem1, gsem2, osem1, osem2):
        wid = lax.axis_index("s") * 2 + lax.axis_index("c")
        base = wid * per_tile

        def start(ci, b):
            off = base + ci * chunk
            pltpu.sync_copy(src_hbm.at[pl.ds(off, chunk)], is_v.at[b])
            pltpu.sync_copy(dst_hbm.at[pl.ds(off, chunk)], id_v.at[b])
            pltpu.async_copy(xl_hbm.at[is_v.at[b]], gl_v.at[b], gsem1.at[b])
            pltpu.async_copy(xr_hbm.at[id_v.at[b]], gr_v.at[b], gsem2.at[b])

        def finish(ci, b):
            off = base + ci * chunk
            pltpu.make_async_copy(xl_hbm.at[is_v.at[b]], gl_v.at[b], gsem1.at[b]).wait()
            pltpu.make_async_copy(xr_hbm.at[id_v.at[b]], gr_v.at[b], gsem2.at[b]).wait()
            pltpu.async_copy(gl_v.at[b], gl_hbm.at[pl.ds(off, chunk)], osem1.at[b])
            pltpu.async_copy(gr_v.at[b], gr_hbm.at[pl.ds(off, chunk)], osem2.at[b])

        def drain(ci, b):
            off = base + ci * chunk
            pltpu.make_async_copy(gl_v.at[b], gl_hbm.at[pl.ds(off, chunk)], osem1.at[b]).wait()
            pltpu.make_async_copy(gr_v.at[b], gr_hbm.at[pl.ds(off, chunk)], osem2.at[b]).wait()

        start(0, 0)

        @pl.loop(0, nch - 1)
        def _(ci):
            b = lax.rem(ci, 2)
            nb = 1 - b
            @pl.when(ci >= 1)
            def _():
                drain(ci - 1, nb)
            start(ci + 1, nb)
            finish(ci, b)

        last = nch - 1
        finish_b = lax.rem(last, 2)
        drain(last - 1, 1 - finish_b)
        finish(last, finish_b)
        drain(last, finish_b)

    return k(xl, xr, src1, dst1)


def _alpha_body(gl_ref, gr_ref, attb_ref, alpha_ref, gmax_ref, *, be, packed, col):
    i = pl.program_id(0)
    t = _unpack_bf16_pairs(gl_ref[...]) + _unpack_bf16_pairs(gr_ref[...])
    if packed == "pair":
        t = t[:, col * 128:(col + 1) * 128]
    t = jnp.where(t > 0, t, NEG_SLOPE * t)
    a = lax.dot_general(t, attb_ref[...], (((1,), (0,)), ((), ())),
                        preferred_element_type=jnp.float32)
    rows = i * be + lax.broadcasted_iota(jnp.int32, a.shape, 0)
    a = jnp.where(rows < ET, a, -1e30)
    alpha_ref[...] = a
    bm8 = jnp.broadcast_to(jnp.max(a, axis=0, keepdims=True), (8, 16))

    @pl.when(i == 0)
    def _():
        gmax_ref[...] = bm8

    @pl.when(i > 0)
    def _():
        gmax_ref[...] = jnp.maximum(gmax_ref[...], bm8)


def _edge_alpha(gl, gr, attb, packed, col):
    hw = gl.shape[1] // 2 if packed == "each" else gl.shape[1]
    bcol = col if packed == "each" else 0
    be = 1024
    return pl.pallas_call(
        functools.partial(_alpha_body, be=be, packed=packed, col=col),
        grid=(EP // be,),
        in_specs=[
            pl.BlockSpec((be, hw), lambda i, c=bcol: (i, c)),
            pl.BlockSpec((be, hw), lambda i, c=bcol: (i, c)),
            pl.BlockSpec((attb.shape[0], 16), lambda i: (0, 0)),
        ],
        out_specs=(
            pl.BlockSpec((be, 16), lambda i: (i, 0)),
            pl.BlockSpec((8, 16), lambda i: (0, 0)),
        ),
        out_shape=(
            jax.ShapeDtypeStruct((EP, 16), jnp.float32),
            jax.ShapeDtypeStruct((8, 16), jnp.float32),
        ),
    )(gl, gr, attb)


def _msg_body(gl_ref, alpha_ref, gmax_ref, *m_refs, heads, packed, col):
    g = jnp.max(gmax_ref[...], axis=0, keepdims=True)
    w = jnp.exp(alpha_ref[...] - g)
    gl = _unpack_bf16_pairs(gl_ref[...])
    if packed == "pair":
        gl = gl[:, col * 128:(col + 1) * 128]
    for q in range(heads):
        m_refs[q][...] = gl[:, q * 128:(q + 1) * 128] * w[:, q:q + 1]
    pad = jnp.zeros((w.shape[0], SLICE_W - 16), jnp.float32)
    m_refs[heads][...] = jnp.concatenate([w, pad], axis=1)


def _edge_messages(gl, alpha, gmax, heads, packed, col):
    hw = gl.shape[1] // 2 if packed == "each" else gl.shape[1]
    bcol = col if packed == "each" else 0
    be = 1024
    return pl.pallas_call(
        functools.partial(_msg_body, heads=heads, packed=packed, col=col),
        grid=(EP // be,),
        in_specs=[
            pl.BlockSpec((be, hw), lambda i, c=bcol: (i, c)),
            pl.BlockSpec((be, 16), lambda i: (i, 0)),
            pl.BlockSpec((8, 16), lambda i: (0, 0)),
        ],
        out_specs=tuple(pl.BlockSpec((be, SLICE_W), lambda i: (i, 0))
                        for _ in range(heads + 1)),
        out_shape=tuple(jax.ShapeDtypeStruct((EP, SLICE_W), jnp.float32)
                        for _ in range(heads + 1)),
    )(gl, alpha, gmax)


def _scatter_accumulate(dst2, m_slices, zeros_nw):
    nsl = len(m_slices)
    win = 128

    @functools.partial(
        pl.kernel,
        mesh=_mesh(),
        out_type=jax.ShapeDtypeStruct((nsl, 2, NACC, SLICE_W), jnp.float32),
        scratch_types=[pltpu.VMEM_SHARED((NACC, SLICE_W), jnp.float32)],
    )
    def k(dst_hbm, *rest):
        m_hbms = rest[:nsl]
        z_hbm = rest[nsl]
        p_hbm = rest[nsl + 1]
        acc = rest[nsl + 2]
        cid = lax.axis_index("c")
        sid = lax.axis_index("s")
        row0 = sid * ROWS_PER_TILE

        def body(id_v, rows_v):
            pltpu.sync_copy(rows_v, acc.at[id_v.at[0]], add=True)

        for q in range(nsl):
            pltpu.sync_copy(z_hbm.at[pl.ds(row0, ROWS_PER_TILE)],
                            acc.at[pl.ds(row0, ROWS_PER_TILE)])
            plsc.subcore_barrier()
            pltpu.emit_pipeline(
                body,
                grid=(EP // win,),
                in_specs=[
                    pl.BlockSpec((1, win), lambda i: (0, i)),
                    pl.BlockSpec((win, SLICE_W), lambda i: (i, 0)),
                ],
                out_specs=[],
                core_axis_name=("c", "s"),
                dimension_semantics=(pltpu.PARALLEL,),
            )(dst_hbm, m_hbms[q])
            plsc.subcore_barrier()
            pltpu.sync_copy(acc.at[pl.ds(row0, ROWS_PER_TILE)],
                            p_hbm.at[q, cid, pl.ds(row0, ROWS_PER_TILE)])
            plsc.subcore_barrier()

    return k(dst2, *m_slices, zeros_nw)


def _norm_body(p_ref, b_ref, o_ref, *, heads, relu):
    p = p_ref[...]
    ps = p[:, 0] + p[:, 1]
    den = ps[heads, :, :heads]
    parts = [ps[h] / (den[:, h:h + 1] + 1e-30) for h in range(heads)]
    out = jnp.concatenate(parts, axis=1) + b_ref[...]
    o_ref[...] = jnp.maximum(out, 0.0) if relu else out


def _normalize(p, bias, heads, relu, qoff):
    bn = 1000
    hd = heads * 128
    return pl.pallas_call(
        functools.partial(_norm_body, heads=heads, relu=relu),
        grid=(N // bn,),
        in_specs=[
            pl.BlockSpec((heads + 1, 2, bn, SLICE_W),
                         lambda i, qoff=qoff: (qoff, 0, i, 0)),
            pl.BlockSpec((1, hd), lambda i: (0, 0)),
        ],
        out_specs=pl.BlockSpec((bn, hd), lambda i: (i, 0)),
        out_shape=jax.ShapeDtypeStruct((N, hd), jnp.float32),
    )(p, bias.reshape(1, hd))


def _attb(att, heads):
    hd = heads * 128
    m = jnp.zeros((hd, 16), jnp.float32)
    for h in range(heads):
        m = m.at[h * 128:(h + 1) * 128, h].set(att[h])
    return m


def _gat_pair(xa, xb, wa, wb, atta, attb_, ba, bb, heads, src1, dst1, dst2,
              zeros_nw, relu_a):
    packed = "each" if heads == 4 else "pair"
    xl, xr = _project_pair(xa, xb, wa, wb, packed)
    chunk = 56 if heads == 4 else 128
    gl, gr = _gather_rows(xl, xr, src1, dst1, chunk)
    ma = _attb(atta, heads)
    mb = _attb(attb_, heads)
    alpha_a, gmax_a = _edge_alpha(gl, gr, ma, packed, 0)
    alpha_b, gmax_b = _edge_alpha(gl, gr, mb, packed, 1)
    msl_a = _edge_messages(gl, alpha_a, gmax_a, heads, packed, 0)
    msl_b = _edge_messages(gl, alpha_b, gmax_b, heads, packed, 1)
    p_a = _scatter_accumulate(dst2, list(msl_a), zeros_nw)
    p_b = _scatter_accumulate(dst2, list(msl_b), zeros_nw)
    out_a = _normalize(p_a, ba, heads, relu_a, 0)
    out_b = _normalize(p_b, bb, heads, False, 0)
    return out_a, out_b


def _head_body(s_ref, p_ref, o1_ref, o2_ref):
    s = s_ref[...]
    p = p_ref[...]
    o1_ref[...] = jnp.concatenate([s[:, :S_DIM], p[:, :P_DIM]], axis=-1)
    sp_s = jnp.logaddexp(s[:, S_DIM:], 0.0)
    sp_p = jnp.logaddexp(p[:, P_DIM:], 0.0)
    o2_ref[...] = jnp.concatenate([sp_s + ETA, sp_p + ETA], axis=-1)


def kernel(x, edge_index, Wl_s1, bl_s1, Wr_s1, br_s1, att_s1, b_s1,
           Wl_s2, bl_s2, Wr_s2, br_s2, att_s2, b_s2,
           Wl_p1, bl_p1, Wr_p1, br_p1, att_p1, b_p1,
           Wl_p2, bl_p2, Wr_p2, br_p2, att_p2, b_p2):
    loop = jnp.arange(N, dtype=edge_index.dtype)
    pad = jnp.zeros((EP - ET,), edge_index.dtype)
    src1 = jnp.concatenate([edge_index[0], loop, pad])
    dst1 = jnp.concatenate([edge_index[1], loop, pad])
    dst2 = dst1.reshape(1, EP)
    zeros_nw = jnp.zeros((NACC, SLICE_W), jnp.float32)

    s, p = _gat_pair(x, x, (Wl_s1, bl_s1, Wr_s1, br_s1),
                     (Wl_p1, bl_p1, Wr_p1, br_p1), att_s1, att_p1,
                     b_s1, b_p1, 4, src1, dst1, dst2, zeros_nw, relu_a=True)
    s, p = _gat_pair(s, p, (Wl_s2, bl_s2, Wr_s2, br_s2),
                     (Wl_p2, bl_p2, Wr_p2, br_p2), att_s2, att_p2,
                     b_s2, b_p2, 1, src1, dst1, dst2, zeros_nw, relu_a=False)

    out1, out2 = pl.pallas_call(
        _head_body,
        out_shape=(
            jax.ShapeDtypeStruct((N, S_DIM + P_DIM), jnp.float32),
            jax.ShapeDtypeStruct((N, S_DIM + P_DIM), jnp.float32),
        ),
    )(s, p)
    return (out1, out2)


# speedup vs baseline: 1.4283x; 1.0010x over previous
import functools

import jax
import jax.numpy as jnp
from jax import lax
from jax.experimental import pallas as pl
from jax.experimental.pallas import tpu as pltpu
from jax.experimental.pallas import tpu_sc as plsc

N = 10000
E = 160000
ET = E + N
EP = 172032
IN_DIM = 256
HIDDEN = 128
S_DIM = 64
P_DIM = 64
ETA = 1e-6
NEG_SLOPE = 0.2
SLICE_W = 128
NACC = 10112
ROWS_PER_TILE = NACC // 16

def _mesh():
    return plsc.VectorSubcoreMesh(core_axis_name="c", subcore_axis_name="s")


def _pack_bf16_pairs(v):
    h = v.shape[1] // 2
    lo = v[:, :h].astype(jnp.bfloat16).astype(jnp.float32)
    hi = v[:, h:].astype(jnp.bfloat16).astype(jnp.float32)
    lo_u = lax.bitcast_convert_type(lo, jnp.uint32)
    hi_u = lax.bitcast_convert_type(hi, jnp.uint32)
    packed = lax.shift_right_logical(lo_u, jnp.uint32(16)) | (hi_u & jnp.uint32(0xFFFF0000))
    return lax.bitcast_convert_type(packed, jnp.int32)


def _unpack_bf16_pairs(gi):
    u = lax.bitcast_convert_type(gi, jnp.uint32)
    lo = lax.bitcast_convert_type(lax.shift_left(u, jnp.uint32(16)), jnp.float32)
    hi = lax.bitcast_convert_type(u & jnp.uint32(0xFFFF0000), jnp.float32)
    return jnp.concatenate([lo, hi], axis=1)


def _mm2_body(xa_ref, xb_ref, wla_ref, bla_ref, wra_ref, bra_ref,
              wlb_ref, blb_ref, wrb_ref, brb_ref, xl_ref, xr_ref, *, packed):
    xa = xa_ref[...]
    xb = xb_ref[...]

    def proj(x, w_ref, b_ref):
        return lax.dot_general(x, w_ref[...], (((1,), (0,)), ((), ())),
                               preferred_element_type=jnp.float32) + b_ref[...]

    xla = proj(xa, wla_ref, bla_ref)
    xra = proj(xa, wra_ref, bra_ref)
    xlb = proj(xb, wlb_ref, blb_ref)
    xrb = proj(xb, wrb_ref, brb_ref)
    if packed == "each":
        xl_ref[...] = jnp.concatenate([_pack_bf16_pairs(xla), _pack_bf16_pairs(xlb)], axis=1)
        xr_ref[...] = jnp.concatenate([_pack_bf16_pairs(xra), _pack_bf16_pairs(xrb)], axis=1)
    else:
        xl_ref[...] = _pack_bf16_pairs(jnp.concatenate([xla, xlb], axis=1))
        xr_ref[...] = _pack_bf16_pairs(jnp.concatenate([xra, xrb], axis=1))


def _project_pair(xa, xb, wa, xw_b, packed):
    n = xa.shape[0]
    ina, inb = xa.shape[1], xb.shape[1]
    hd = wa[0].shape[1]
    wout = hd
    odt = jnp.int32
    bm = 2000
    wla, bla, wra, bra = wa
    wlb, blb, wrb, brb = xw_b
    return pl.pallas_call(
        functools.partial(_mm2_body, packed=packed),
        grid=(n // bm,),
        in_specs=[
            pl.BlockSpec((bm, ina), lambda i: (i, 0)),
            pl.BlockSpec((bm, inb), lambda i: (i, 0)),
            pl.BlockSpec((ina, hd), lambda i: (0, 0)),
            pl.BlockSpec((1, hd), lambda i: (0, 0)),
            pl.BlockSpec((ina, hd), lambda i: (0, 0)),
            pl.BlockSpec((1, hd), lambda i: (0, 0)),
            pl.BlockSpec((inb, hd), lambda i: (0, 0)),
            pl.BlockSpec((1, hd), lambda i: (0, 0)),
            pl.BlockSpec((inb, hd), lambda i: (0, 0)),
            pl.BlockSpec((1, hd), lambda i: (0, 0)),
        ],
        out_specs=(
            pl.BlockSpec((bm, wout), lambda i: (i, 0)),
            pl.BlockSpec((bm, wout), lambda i: (i, 0)),
        ),
        out_shape=(
            jax.ShapeDtypeStruct((n, wout), odt),
            jax.ShapeDtypeStruct((n, wout), odt),
        ),
    )(xa, xb, wla, bla.reshape(1, hd), wra, bra.reshape(1, hd),
      wlb, blb.reshape(1, hd), wrb, brb.reshape(1, hd))


def _gather_rows(xl, xr, src1, dst1, chunk):
    hd = xl.shape[1]
    per_tile = EP // 32
    nch = per_tile // chunk

    @functools.partial(
        pl.kernel,
        mesh=_mesh(),
        out_type=(
            jax.ShapeDtypeStruct((EP, hd), xl.dtype),
            jax.ShapeDtypeStruct((EP, hd), xl.dtype),
        ),
        scratch_types=[
            pltpu.VMEM((2, chunk), jnp.int32),
            pltpu.VMEM((2, chunk), jnp.int32),
            pltpu.VMEM((2, chunk, hd), xl.dtype),
            pltpu.VMEM((2, chunk, hd), xl.dtype),
            pltpu.SemaphoreType.DMA((2,)),
            pltpu.SemaphoreType.DMA((2,)),
            pltpu.SemaphoreType.DMA((2,)),
            pltpu.SemaphoreType.DMA((2,)),
        ],
    )
    def k(xl_hbm, xr_hbm, src_hbm, dst_hbm, gl_hbm, gr_hbm,
          is_v, id_v, gl_v, gr_v, gsem1, gsem2, osem1, osem2):
        wid = lax.axis_index("s") * 2 + lax.axis_index("c")
        base = wid * per_tile

        def start(ci, b):
            off = base + ci * chunk
            pltpu.sync_copy(src_hbm.at[pl.ds(off, chunk)], is_v.at[b])
            pltpu.sync_copy(dst_hbm.at[pl.ds(off, chunk)], id_v.at[b])
            pltpu.async_copy(xl_hbm.at[is_v.at[b]], gl_v.at[b], gsem1.at[b])
            pltpu.async_copy(xr_hbm.at[id_v.at[b]], gr_v.at[b], gsem2.at[b])

        def finish(ci, b):
            off = base + ci * chunk
            pltpu.make_async_copy(xl_hbm.at[is_v.at[b]], gl_v.at[b], gsem1.at[b]).wait()
            pltpu.make_async_copy(xr_hbm.at[id_v.at[b]], gr_v.at[b], gsem2.at[b]).wait()
            pltpu.async_copy(gl_v.at[b], gl_hbm.at[pl.ds(off, chunk)], osem1.at[b])
            pltpu.async_copy(gr_v.at[b], gr_hbm.at[pl.ds(off, chunk)], osem2.at[b])

        def drain(ci, b):
            off = base + ci * chunk
            pltpu.make_async_copy(gl_v.at[b], gl_hbm.at[pl.ds(off, chunk)], osem1.at[b]).wait()
            pltpu.make_async_copy(gr_v.at[b], gr_hbm.at[pl.ds(off, chunk)], osem2.at[b]).wait()

        start(0, 0)

        @pl.loop(0, nch - 1)
        def _(ci):
            b = lax.rem(ci, 2)
            nb = 1 - b
            @pl.when(ci >= 1)
            def _():
                drain(ci - 1, nb)
            start(ci + 1, nb)
            finish(ci, b)

        last = nch - 1
        finish_b = lax.rem(last, 2)
        drain(last - 1, 1 - finish_b)
        finish(last, finish_b)
        drain(last, finish_b)

    return k(xl, xr, src1, dst1)


def _edge_body(gl_ref, gr_ref, attba_ref, attbb_ref, *m_refs, heads, be, packed):
    i = pl.program_id(0)
    rows = i * be + lax.broadcasted_iota(jnp.int32, (be, 16), 0)
    mask = rows < ET
    hd = heads * 128
    for j, attb_ref in ((0, attba_ref), (1, attbb_ref)):
        if packed == "each":
            glj = _unpack_bf16_pairs(gl_ref[...][:, j * 256:(j + 1) * 256])
            grj = _unpack_bf16_pairs(gr_ref[...][:, j * 256:(j + 1) * 256])
        else:
            glj = _unpack_bf16_pairs(gl_ref[...])[:, j * 128:(j + 1) * 128]
            grj = _unpack_bf16_pairs(gr_ref[...])[:, j * 128:(j + 1) * 128]
        t = glj + grj
        t = jnp.where(t > 0, t, NEG_SLOPE * t)
        a = lax.dot_general(t, attb_ref[...], (((1,), (0,)), ((), ())),
                            preferred_element_type=jnp.float32)
        w = jnp.where(mask, jnp.exp(a), 0.0)
        for q in range(heads):
            m_refs[j * (heads + 1) + q][...] = glj[:, q * 128:(q + 1) * 128] * w[:, q:q + 1]
        pad = jnp.zeros((be, SLICE_W - 16), jnp.float32)
        m_refs[j * (heads + 1) + heads][...] = jnp.concatenate([w, pad], axis=1)


def _edge_pair(gl, gr, attba, attbb, heads, packed):
    wfull = gl.shape[1]
    be = 1024
    nsl = heads + 1
    return pl.pallas_call(
        functools.partial(_edge_body, heads=heads, be=be, packed=packed),
        grid=(EP // be,),
        in_specs=[
            pl.BlockSpec((be, wfull), lambda i: (i, 0)),
            pl.BlockSpec((be, wfull), lambda i: (i, 0)),
            pl.BlockSpec((attba.shape[0], 16), lambda i: (0, 0)),
            pl.BlockSpec((attbb.shape[0], 16), lambda i: (0, 0)),
        ],
        out_specs=tuple(pl.BlockSpec((be, SLICE_W), lambda i: (i, 0))
                        for _ in range(2 * nsl)),
        out_shape=tuple(jax.ShapeDtypeStruct((EP, SLICE_W), jnp.float32)
                        for _ in range(2 * nsl)),
    )(gl, gr, attba, attbb)


def _scatter_accumulate(dst2, m_slices, zeros_nw):
    nsl = len(m_slices)
    win = 128

    @functools.partial(
        pl.kernel,
        mesh=_mesh(),
        out_type=jax.ShapeDtypeStruct((nsl, 2, NACC, SLICE_W), jnp.float32),
        scratch_types=[pltpu.VMEM_SHARED((NACC, SLICE_W), jnp.float32)],
    )
    def k(dst_hbm, *rest):
        m_hbms = rest[:nsl]
        z_hbm = rest[nsl]
        p_hbm = rest[nsl + 1]
        acc = rest[nsl + 2]
        cid = lax.axis_index("c")
        sid = lax.axis_index("s")
        row0 = sid * ROWS_PER_TILE

        def body(id_v, rows_v):
            pltpu.sync_copy(rows_v, acc.at[id_v.at[0]], add=True)

        for q in range(nsl):
            pltpu.sync_copy(z_hbm.at[pl.ds(row0, ROWS_PER_TILE)],
                            acc.at[pl.ds(row0, ROWS_PER_TILE)])
            plsc.subcore_barrier()
            pltpu.emit_pipeline(
                body,
                grid=(EP // win,),
                in_specs=[
                    pl.BlockSpec((1, win), lambda i: (0, i)),
                    pl.BlockSpec((win, SLICE_W), lambda i: (i, 0)),
                ],
                out_specs=[],
                core_axis_name=("c", "s"),
                dimension_semantics=(pltpu.PARALLEL,),
            )(dst_hbm, m_hbms[q])
            plsc.subcore_barrier()
            pltpu.sync_copy(acc.at[pl.ds(row0, ROWS_PER_TILE)],
                            p_hbm.at[q, cid, pl.ds(row0, ROWS_PER_TILE)])
            plsc.subcore_barrier()

    return k(dst2, *m_slices, zeros_nw)


def _norm_body(p_ref, b_ref, o_ref, *, heads, relu):
    p = p_ref[...]
    ps = p[:, 0] + p[:, 1]
    den = ps[heads, :, :heads]
    parts = [ps[h] / (den[:, h:h + 1] + 1e-30) for h in range(heads)]
    out = jnp.concatenate(parts, axis=1) + b_ref[...]
    o_ref[...] = jnp.maximum(out, 0.0) if relu else out


def _normalize(p, bias, heads, relu, qoff):
    bn = 1000
    hd = heads * 128
    return pl.pallas_call(
        functools.partial(_norm_body, heads=heads, relu=relu),
        grid=(N // bn,),
        in_specs=[
            pl.BlockSpec((heads + 1, 2, bn, SLICE_W),
                         lambda i, qoff=qoff: (qoff, 0, i, 0)),
            pl.BlockSpec((1, hd), lambda i: (0, 0)),
        ],
        out_specs=pl.BlockSpec((bn, hd), lambda i: (i, 0)),
        out_shape=jax.ShapeDtypeStruct((N, hd), jnp.float32),
    )(p, bias.reshape(1, hd))


def _attb(att, heads):
    hd = heads * 128
    m = jnp.zeros((hd, 16), jnp.float32)
    for h in range(heads):
        m = m.at[h * 128:(h + 1) * 128, h].set(att[h])
    return m


def _gat_pair(xa, xb, wa, wb, atta, attb_, ba, bb, heads, src1, dst1, dst2,
              zeros_nw, relu_a):
    packed = "each" if heads == 4 else "pair"
    xl, xr = _project_pair(xa, xb, wa, wb, packed)
    chunk = 56 if heads == 4 else 128
    gl, gr = _gather_rows(xl, xr, src1, dst1, chunk)
    ma = _attb(atta, heads)
    mb = _attb(attb_, heads)
    m_all = _edge_pair(gl, gr, ma, mb, heads, packed)
    msl_a = list(m_all[:heads + 1])
    msl_b = list(m_all[heads + 1:])
    p_a = _scatter_accumulate(dst2, msl_a, zeros_nw)
    p_b = _scatter_accumulate(dst2, msl_b, zeros_nw)
    out_a = _normalize(p_a, ba, heads, relu_a, 0)
    out_b = _normalize(p_b, bb, heads, False, 0)
    return out_a, out_b


def _head_body(s_ref, p_ref, o1_ref, o2_ref):
    s = s_ref[...]
    p = p_ref[...]
    o1_ref[...] = jnp.concatenate([s[:, :S_DIM], p[:, :P_DIM]], axis=-1)
    sp_s = jnp.logaddexp(s[:, S_DIM:], 0.0)
    sp_p = jnp.logaddexp(p[:, P_DIM:], 0.0)
    o2_ref[...] = jnp.concatenate([sp_s + ETA, sp_p + ETA], axis=-1)


def kernel(x, edge_index, Wl_s1, bl_s1, Wr_s1, br_s1, att_s1, b_s1,
           Wl_s2, bl_s2, Wr_s2, br_s2, att_s2, b_s2,
           Wl_p1, bl_p1, Wr_p1, br_p1, att_p1, b_p1,
           Wl_p2, bl_p2, Wr_p2, br_p2, att_p2, b_p2):
    loop = jnp.arange(N, dtype=edge_index.dtype)
    pad = jnp.zeros((EP - ET,), edge_index.dtype)
    src1 = jnp.concatenate([edge_index[0], loop, pad])
    dst1 = jnp.concatenate([edge_index[1], loop, pad])
    dst2 = dst1.reshape(1, EP)
    zeros_nw = jnp.zeros((NACC, SLICE_W), jnp.float32)

    s, p = _gat_pair(x, x, (Wl_s1, bl_s1, Wr_s1, br_s1),
                     (Wl_p1, bl_p1, Wr_p1, br_p1), att_s1, att_p1,
                     b_s1, b_p1, 4, src1, dst1, dst2, zeros_nw, relu_a=True)
    s, p = _gat_pair(s, p, (Wl_s2, bl_s2, Wr_s2, br_s2),
                     (Wl_p2, bl_p2, Wr_p2, br_p2), att_s2, att_p2,
                     b_s2, b_p2, 1, src1, dst1, dst2, zeros_nw, relu_a=False)

    out1, out2 = pl.pallas_call(
        _head_body,
        out_shape=(
            jax.ShapeDtypeStruct((N, S_DIM + P_DIM), jnp.float32),
            jax.ShapeDtypeStruct((N, S_DIM + P_DIM), jnp.float32),
        ),
    )(s, p)
    return (out1, out2)
